# baseline (speedup 1.0000x reference)
import jax
import jax.numpy as jnp
from jax import lax
from jax.experimental import pallas as pl
from jax.experimental.pallas import tpu as pltpu
from jax.experimental.pallas import tpu_sc as plsc

_N_IDX = 8


def _gather_body(idx_hbm, buf_hbm, out_hbm, idx_v, val_v, sem):
    cid = lax.axis_index("c")
    sid = lax.axis_index("s")

    @pl.when(jnp.logical_and(cid == 0, sid == 0))
    def _():
        pltpu.sync_copy(idx_hbm, idx_v)
        pltpu.async_copy(buf_hbm.at[idx_v], val_v, sem).wait()
        pltpu.sync_copy(val_v, out_hbm)


def kernel(x, buffer, buffer_index):
    del x
    idx = buffer_index.astype(jnp.int32)
    idx_list = jnp.broadcast_to(idx, (_N_IDX,))
    mesh = plsc.VectorSubcoreMesh(core_axis_name="c", subcore_axis_name="s")
    gather = pl.kernel(
        _gather_body,
        out_type=jax.ShapeDtypeStruct((_N_IDX,), jnp.float32),
        mesh=mesh,
        scratch_types=[
            pltpu.VMEM((_N_IDX,), jnp.int32),
            pltpu.VMEM((_N_IDX,), jnp.float32),
            pltpu.SemaphoreType.DMA,
        ],
    )
    out = gather(idx_list, buffer)
    return out[0]


# speedup vs baseline: 1.0746x; 1.0746x over previous
import jax
import jax.numpy as jnp
from jax.experimental import pallas as pl
from jax.experimental.pallas import tpu as pltpu
from jax.experimental.pallas import tpu_sc as plsc

_N_IDX = 8


def _gather_body(idx_hbm, buf_hbm, out_hbm, idx_v, val_v, sem):
    pltpu.sync_copy(idx_hbm, idx_v)
    pltpu.async_copy(buf_hbm.at[idx_v], val_v, sem).wait()
    pltpu.sync_copy(val_v, out_hbm)


def kernel(x, buffer, buffer_index):
    del x
    idx = buffer_index.astype(jnp.int32)
    idx_list = jnp.broadcast_to(idx, (_N_IDX,))
    mesh = plsc.VectorSubcoreMesh(
        core_axis_name="c", subcore_axis_name="s", num_cores=1, num_subcores=1
    )
    gather = pl.kernel(
        _gather_body,
        out_type=jax.ShapeDtypeStruct((_N_IDX,), jnp.float32),
        mesh=mesh,
        scratch_types=[
            pltpu.VMEM((_N_IDX,), jnp.int32),
            pltpu.VMEM((_N_IDX,), jnp.float32),
            pltpu.SemaphoreType.DMA,
        ],
    )
    out = gather(idx_list, buffer)
    return out[0]
